# Initial kernel scaffold; baseline (speedup 1.0000x reference)
import functools

import jax
import jax.numpy as jnp
from jax import lax
from jax.experimental import pallas as pl
from jax.experimental.pallas import tpu as pltpu
from jax.experimental.pallas import tpu_sc as plsc

N_NODES = 50000
N_EDGES = 800000
N_GRAPHS = 256
IN_DIM = 74
HID = 256
EHID = 32

NP = 51200
GPAD = 320
EPT = 25000
EPT_SC = 50000
EBATCH = 10000
FB = 128

_i32 = jnp.int32
_f32 = jnp.float32


def _iota16():
    return lax.iota(_i32, 16)


def _mesh():
    return plsc.VectorSubcoreMesh(core_axis_name="c", subcore_axis_name="s")


def _copy128(src_ref, src_off, dst_ref):
    for t in range(8):
        dst_ref[16 * t:16 * t + 16] = src_ref[pl.ds(src_off + 16 * t, 16)]


def _hist_scan(ibuf, dmabuf, ones_v, hist, n_idx, sent_base):
    nb_full = n_idx // FB
    rem = n_idx - nb_full * FB

    def body(j, carry):
        _copy128(ibuf, FB * j, dmabuf)
        pltpu.sync_copy(ones_v, hist.at[dmabuf], add=True)
        return carry

    lax.fori_loop(0, nb_full, body, jnp.int32(0))
    if rem > 0:
        for t in range(8):
            lane0 = 16 * t
            v = ibuf[pl.ds(FB * nb_full + lane0, 16)]
            sent = sent_base + _iota16()
            if lane0 + 16 <= rem:
                dmabuf[16 * t:16 * t + 16] = v
            elif lane0 >= rem:
                dmabuf[16 * t:16 * t + 16] = sent
            else:
                m = _iota16() < (rem - lane0)
                dmabuf[16 * t:16 * t + 16] = jnp.where(m, v, sent)
        pltpu.sync_copy(ones_v, hist.at[dmabuf], add=True)


def _deg_kernel(src_hbm, dst_hbm, gid_hbm, zflat_hbm,
                dop, dip, cntp,
                hist_o, hist_i, hist_g,
                ibuf, dmabuf, ones_v):
    sc = lax.axis_index("c")
    sub = lax.axis_index("s")
    for t in range(8):
        ones_v[16 * t:16 * t + 16] = jnp.ones((16,), _f32)
    pltpu.sync_copy(zflat_hbm, hist_o.at[pl.ds(sub * 3200, 3200)])
    pltpu.sync_copy(zflat_hbm, hist_i.at[pl.ds(sub * 3200, 3200)])

    @pl.when(sub == 0)
    def _():
        pltpu.sync_copy(zflat_hbm.at[pl.ds(0, 16)], hist_o.at[pl.ds(NP, 16)])
        pltpu.sync_copy(zflat_hbm.at[pl.ds(0, 16)], hist_i.at[pl.ds(NP, 16)])
        pltpu.sync_copy(zflat_hbm.at[pl.ds(0, GPAD)], hist_g)

    plsc.subcore_barrier()

    ebase = (sc * 16 + sub) * EPT
    pltpu.sync_copy(src_hbm.at[pl.ds(ebase, EPT)], ibuf.at[pl.ds(0, EPT)])
    _hist_scan(ibuf, dmabuf, ones_v, hist_o, EPT, NP)
    pltpu.sync_copy(dst_hbm.at[pl.ds(ebase, EPT)], ibuf.at[pl.ds(0, EPT)])
    _hist_scan(ibuf, dmabuf, ones_v, hist_i, EPT, NP)

    gbase = sc * (NP // 2) + sub * 1600
    pltpu.sync_copy(gid_hbm.at[pl.ds(gbase, 1600)], ibuf.at[pl.ds(0, 1600)])
    _hist_scan(ibuf, dmabuf, ones_v, hist_g, 1600, N_GRAPHS)

    plsc.subcore_barrier()
    pltpu.sync_copy(hist_o.at[pl.ds(sub * 3200, 3200)],
                    dop.at[sc, pl.ds(sub * 3200, 3200)])
    pltpu.sync_copy(hist_i.at[pl.ds(sub * 3200, 3200)],
                    dip.at[sc, pl.ds(sub * 3200, 3200)])

    @pl.when(sub == 0)
    def _():
        pltpu.sync_copy(hist_g, cntp.at[sc])


def _run_deg(src, dst, gid_pad, zflat):
    k = functools.partial(
        pl.kernel,
        out_type=[
            jax.ShapeDtypeStruct((2, NP), _f32),
            jax.ShapeDtypeStruct((2, NP), _f32),
            jax.ShapeDtypeStruct((2, GPAD), _f32),
        ],
        mesh=_mesh(),
        scratch_types=[
            pltpu.VMEM_SHARED((NP + 16,), _f32),
            pltpu.VMEM_SHARED((NP + 16,), _f32),
            pltpu.VMEM_SHARED((GPAD,), _f32),
            pltpu.VMEM((FB * (EPT // FB) + FB,), _i32),
            pltpu.VMEM((FB,), _i32),
            pltpu.VMEM((FB,), _f32),
        ],
    )(_deg_kernel)
    return k(src, dst, gid_pad, zflat)


def _agg_flush(table_hbm, acc, csrc_st, cdst_st, csrc_dma, cdst_dma,
               rowbuf, sem, off):
    for t in range(8):
        csrc_dma[16 * t:16 * t + 16] = csrc_st[16 * t:16 * t + 16]
        cdst_dma[16 * t:16 * t + 16] = cdst_st[16 * t:16 * t + 16]
    pltpu.async_copy(table_hbm.at[csrc_dma], rowbuf, sem).wait()
    pltpu.sync_copy(rowbuf, acc.at[cdst_dma], add=True)
    ts = csrc_st[pl.ds(FB, 16)]
    td = cdst_st[pl.ds(FB, 16)]
    csrc_st[pl.ds(0, 16)] = ts
    cdst_st[pl.ds(0, 16)] = td
    return off - FB


def _make_agg_kernel(D, R, n_chunks):
    cpc = n_chunks // 2
    n_batches = EPT_SC // EBATCH
    vregs = EBATCH // 16
    rpt = R // 16

    def body(table_hbm, src_hbm, dst_hbm, zr_hbm, out_hbm,
             acc, sbuf, dbuf, csrc_st, cdst_st, csrc_dma, cdst_dma,
             rowbuf, zrow, sem):
        sc = lax.axis_index("c")
        sub = lax.axis_index("s")
        pltpu.sync_copy(zr_hbm, zrow)
        ebase0 = sub * EPT_SC

        for i in range(cpc):
            c = 2 * i + sc
            lo = c * R
            hi = lo + R

            def zbody(jj, carry):
                pltpu.sync_copy(zrow, acc.at[pl.ds(sub * rpt + 16 * jj, 16)])
                return carry
            lax.fori_loop(0, rpt // 16, zbody, jnp.int32(0))

            @pl.when(sub == 0)
            def _():
                pltpu.sync_copy(zrow, acc.at[pl.ds(R, 16)])

            plsc.subcore_barrier()

            off = jnp.int32(0)
            for b in range(n_batches):
                eb = ebase0 + b * EBATCH
                pltpu.sync_copy(src_hbm.at[pl.ds(eb, EBATCH)], sbuf)
                pltpu.sync_copy(dst_hbm.at[pl.ds(eb, EBATCH)], dbuf)

                def vbody(v, o):
                    s = sbuf[pl.ds(v * 16, 16)]
                    d = dbuf[pl.ds(v * 16, 16)]
                    m = (d >= lo) & (d < hi)
                    plsc.store_compressed(csrc_st.at[pl.ds(o, 16)], s, m)
                    plsc.store_compressed(cdst_st.at[pl.ds(o, 16)], d - lo, m)
                    o = o + jnp.sum(jnp.where(m, 1, 0).astype(_i32))
                    return lax.cond(
                        o >= FB,
                        lambda oo: _agg_flush(table_hbm, acc, csrc_st, cdst_st,
                                              csrc_dma, cdst_dma, rowbuf, sem,
                                              oo),
                        lambda oo: oo,
                        o)

                off = lax.fori_loop(0, vregs, vbody, off)

            for t in range(8):
                lane0 = 16 * t
                mv = (lane0 + _iota16()) < off
                sv = csrc_st[16 * t:16 * t + 16]
                dv = cdst_st[16 * t:16 * t + 16]
                csrc_dma[16 * t:16 * t + 16] = jnp.where(mv, sv, _iota16())
                cdst_dma[16 * t:16 * t + 16] = jnp.where(mv, dv, R + _iota16())
            pltpu.async_copy(table_hbm.at[csrc_dma], rowbuf, sem).wait()
            pltpu.sync_copy(rowbuf, acc.at[cdst_dma], add=True)

            plsc.subcore_barrier()
            pltpu.sync_copy(
                acc.at[pl.ds(sub * rpt, rpt)],
                out_hbm.at[pl.ds(c * R + sub * rpt, rpt)])
            plsc.subcore_barrier()

    return body


def _run_agg(table, src, dst, zr, D, R, n_chunks):
    body = _make_agg_kernel(D, R, n_chunks)
    k = functools.partial(
        pl.kernel,
        out_type=jax.ShapeDtypeStruct((NP, D), _f32),
        mesh=_mesh(),
        scratch_types=[
            pltpu.VMEM_SHARED((R + 16, D), _f32),
            pltpu.VMEM((EBATCH,), _i32),
            pltpu.VMEM((EBATCH,), _i32),
            pltpu.VMEM((FB + 16,), _i32),
            pltpu.VMEM((FB + 16,), _i32),
            pltpu.VMEM((FB,), _i32),
            pltpu.VMEM((FB,), _i32),
            pltpu.VMEM((FB, D), _f32),
            pltpu.VMEM((16, D), _f32),
            pltpu.SemaphoreType.DMA,
        ],
    )(body)
    return k(table, src, dst, zr)


def _pool_kernel(h2_hbm, gid_hbm, zr_hbm, outp,
                 acc, rbuf, gbuf, gdma, zrow):
    sc = lax.axis_index("c")
    sub = lax.axis_index("s")
    pltpu.sync_copy(zr_hbm, zrow)
    pltpu.sync_copy(zrow, acc.at[pl.ds(sub * 20, 16)])
    pltpu.sync_copy(zrow.at[pl.ds(0, 4)], acc.at[pl.ds(sub * 20 + 16, 4)])
    plsc.subcore_barrier()

    rbase = sc * (NP // 2) + sub * 1600
    pltpu.sync_copy(gid_hbm.at[pl.ds(rbase, 1600)], gbuf)
    for b in range(20):
        pltpu.sync_copy(h2_hbm.at[pl.ds(rbase + 80 * b, 80)], rbuf)
        for t in range(5):
            gdma[16 * t:16 * t + 16] = gbuf[pl.ds(80 * b + 16 * t, 16)]
        pltpu.sync_copy(rbuf, acc.at[gdma], add=True)

    plsc.subcore_barrier()
    pltpu.sync_copy(acc.at[pl.ds(sub * 20, 20)],
                    outp.at[sc, pl.ds(sub * 20, 20)])


def _run_pool(h2, gid_pad, zr256):
    k = functools.partial(
        pl.kernel,
        out_type=jax.ShapeDtypeStruct((2, GPAD, HID), _f32),
        mesh=_mesh(),
        scratch_types=[
            pltpu.VMEM_SHARED((GPAD, HID), _f32),
            pltpu.VMEM((80, HID), _f32),
            pltpu.VMEM((1600,), _i32),
            pltpu.VMEM((80,), _i32),
            pltpu.VMEM((16, HID), _f32),
        ],
    )(_pool_kernel)
    return k(h2, gid_pad, zr256)


_BLK = 512
_NBLK = NP // _BLK


def _t1_body(nf_ref, do0, do1, di0, di1, xn_ref, no_ref, ni_ref):
    do = do0[...] + do1[...]
    di = di0[...] + di1[...]
    no = jnp.where(do > 0.0, lax.rsqrt(jnp.maximum(do, 1e-30)), 0.0)
    ni = jnp.where(di > 0.0, lax.rsqrt(jnp.maximum(di, 1e-30)), 0.0)
    xn_ref[...] = nf_ref[...] * no[:, None]
    no_ref[...] = no
    ni_ref[...] = ni


def _run_t1(nf_p, dop, dip):
    return pl.pallas_call(
        _t1_body,
        grid=(_NBLK,),
        in_specs=[
            pl.BlockSpec((_BLK, 80), lambda i: (i, 0)),
            pl.BlockSpec((_BLK,), lambda i: (i,)),
            pl.BlockSpec((_BLK,), lambda i: (i,)),
            pl.BlockSpec((_BLK,), lambda i: (i,)),
            pl.BlockSpec((_BLK,), lambda i: (i,)),
        ],
        out_specs=[
            pl.BlockSpec((_BLK, 80), lambda i: (i, 0)),
            pl.BlockSpec((_BLK,), lambda i: (i,)),
            pl.BlockSpec((_BLK,), lambda i: (i,)),
        ],
        out_shape=[
            jax.ShapeDtypeStruct((NP, 80), _f32),
            jax.ShapeDtypeStruct((NP,), _f32),
            jax.ShapeDtypeStruct((NP,), _f32),
        ],
    )(nf_p, dop[0], dop[1], dip[0], dip[1])


def _t2_body(agg_ref, ni_ref, no_ref, w_ref, b_ref, out_ref):
    a = agg_ref[...] * ni_ref[...][:, None]
    h = jnp.maximum(jnp.dot(a, w_ref[...],
                            preferred_element_type=_f32) + b_ref[...], 0.0)
    out_ref[...] = h * no_ref[...][:, None]


def _run_t2(agg1, ni, no, w, b):
    return pl.pallas_call(
        _t2_body,
        grid=(_NBLK,),
        in_specs=[
            pl.BlockSpec((_BLK, 80), lambda i: (i, 0)),
            pl.BlockSpec((_BLK,), lambda i: (i,)),
            pl.BlockSpec((_BLK,), lambda i: (i,)),
            pl.BlockSpec((80, HID), lambda i: (0, 0)),
            pl.BlockSpec((1, HID), lambda i: (0, 0)),
        ],
        out_specs=pl.BlockSpec((_BLK, HID), lambda i: (i, 0)),
        out_shape=jax.ShapeDtypeStruct((NP, HID), _f32),
    )(agg1, ni, no, w, b)


def _t3_body(agg_ref, ni_ref, w_ref, b_ref, out_ref):
    a = agg_ref[...] * ni_ref[...][:, None]
    out_ref[...] = jnp.maximum(
        jnp.dot(a, w_ref[...], preferred_element_type=_f32) + b_ref[...], 0.0)


def _run_t3(agg2, ni, w, b):
    return pl.pallas_call(
        _t3_body,
        grid=(_NBLK,),
        in_specs=[
            pl.BlockSpec((_BLK, HID), lambda i: (i, 0)),
            pl.BlockSpec((_BLK,), lambda i: (i,)),
            pl.BlockSpec((HID, HID), lambda i: (0, 0)),
            pl.BlockSpec((1, HID), lambda i: (0, 0)),
        ],
        out_specs=pl.BlockSpec((_BLK, HID), lambda i: (i, 0)),
        out_shape=jax.ShapeDtypeStruct((NP, HID), _f32),
    )(agg2, ni, w, b)


def _t4_body(pp_ref, c0_ref, c1_ref, ihb_ref, iff_ref, ssrc_ref, sdst_ref,
             wproj_ref, bproj_ref, we1_ref, be1_ref, w2r_ref, be2r_ref,
             bnn_ref, wih_ref, whh_ref, bih_ref, bhh_ref,
             wr1_ref, br1_ref, wr2_ref, br2_ref, wr3_ref, br3_ref,
             out_ref):
    pool = pp_ref[0] + pp_ref[1]
    cnt = (c0_ref[...] + c1_ref[...])[:N_GRAPHS]
    g = pool[:N_GRAPHS] / jnp.clip(cnt, 1.0, None)[:, None]
    gm = jnp.concatenate([g, ihb_ref[...]], axis=1)
    gm2 = jnp.concatenate([gm, gm], axis=0)
    nf = jnp.maximum(
        jnp.dot(gm2, wproj_ref[...], preferred_element_type=_f32)
        + bproj_ref[...], 0.0)
    eh = jnp.maximum(
        jnp.dot(iff_ref[...], we1_ref[...], preferred_element_type=_f32)
        + be1_ref[...], 0.0)
    iot = lax.broadcasted_iota(_i32, (4 * N_GRAPHS, 2 * N_GRAPHS), 1)
    oh_s = (ssrc_ref[...] == iot).astype(_f32)
    nfs = jnp.dot(oh_s, nf, preferred_element_type=_f32)
    msg = jnp.dot(nfs, be2r_ref[...], preferred_element_type=_f32)
    for kk in range(EHID):
        msg = msg + eh[:, kk:kk + 1] * jnp.dot(
            nfs, w2r_ref[kk], preferred_element_type=_f32)
    oh_d = (sdst_ref[...] == iot).astype(_f32)
    aggm = lax.dot_general(oh_d, msg, (((0,), (0,)), ((), ())),
                           preferred_element_type=_f32)
    nf2 = jnp.maximum(aggm + bnn_ref[...], 0.0)
    gi = lax.dot_general(nf2, wih_ref[...], (((1,), (1,)), ((), ())),
                         preferred_element_type=_f32) + bih_ref[...]
    gh = lax.dot_general(nf, whh_ref[...], (((1,), (1,)), ((), ())),
                         preferred_element_type=_f32) + bhh_ref[...]
    r = jax.nn.sigmoid(gi[:, :HID] + gh[:, :HID])
    z = jax.nn.sigmoid(gi[:, HID:2 * HID] + gh[:, HID:2 * HID])
    n = jnp.tanh(gi[:, 2 * HID:] + r * gh[:, 2 * HID:])
    ghf = (1.0 - z) * n + z * nf
    o = jnp.maximum(jnp.dot(ghf, wr1_ref[...],
                            preferred_element_type=_f32) + br1_ref[...], 0.0)
    o = jnp.maximum(jnp.dot(o, wr2_ref[...],
                            preferred_element_type=_f32) + br2_ref[...], 0.0)
    o = jnp.dot(o, wr3_ref[...], preferred_element_type=_f32) + br3_ref[...]
    out_ref[...] = 0.5 * (o[:N_GRAPHS] + o[N_GRAPHS:])


def _run_t4(pp, cntp, ihb, iff, ssrc, sdst, wproj, bproj, we1, be1, w2r,
            be2r, bnn, wih, whh, bih, bhh, wr1, br1, wr2, br2, wr3, br3):
    return pl.pallas_call(
        _t4_body,
        out_shape=jax.ShapeDtypeStruct((N_GRAPHS, 1), _f32),
    )(pp, cntp[0], cntp[1], ihb, iff, ssrc, sdst, wproj, bproj, we1, be1,
      w2r, be2r, bnn, wih, whh, bih, bhh, wr1, br1, wr2, br2, wr3, br3)


def kernel(node_feats, edge_index, graph_id, inter_hb, be_salt, be_ps, ip,
           solv_edge_index, W_c1, b_c1, W_c2, b_c2, W_proj, b_proj,
           W_e1, b_e1, W_e2, b_e2, b_nn, W_ih, W_hh, b_ih, b_hh,
           W_r1, b_r1, W_r2, b_r2, W_r3, b_r3):
    src = edge_index[0]
    dst = edge_index[1]
    gid_pad = jnp.pad(graph_id, (0, NP - N_NODES), constant_values=N_GRAPHS)
    nf_p = jnp.pad(node_feats, ((0, NP - N_NODES), (0, 80 - IN_DIM)))
    wc1_p = jnp.pad(W_c1, ((0, 80 - IN_DIM), (0, 0)))
    zflat = jnp.zeros((3200,), _f32)
    zr80 = jnp.zeros((16, 80), _f32)
    zr256 = jnp.zeros((16, HID), _f32)

    dop, dip, cntp = _run_deg(src, dst, gid_pad, zflat)
    xn, no, ni = _run_t1(nf_p, dop, dip)
    agg1 = _run_agg(xn, src, dst, zr80, 80, 12800, 4)
    h1n = _run_t2(agg1, ni, no, wc1_p, b_c1.reshape(1, HID))
    agg2 = _run_agg(h1n, src, dst, zr256, HID, 6400, 8)
    h2 = _run_t3(agg2, ni, W_c2, b_c2.reshape(1, HID))
    pp = _run_pool(h2, gid_pad, zr256)

    iff = jnp.concatenate([inter_hb, be_salt, be_ps, ip], axis=0)[:, None]
    ssrc = solv_edge_index[0][:, None]
    sdst = solv_edge_index[1][:, None]
    out = _run_t4(pp, cntp, inter_hb[:, None], iff, ssrc, sdst,
                  W_proj, b_proj.reshape(1, HID),
                  W_e1, b_e1.reshape(1, EHID),
                  W_e2.reshape(EHID, HID, HID), b_e2.reshape(HID, HID),
                  b_nn.reshape(1, HID), W_ih, W_hh,
                  b_ih.reshape(1, 3 * HID), b_hh.reshape(1, 3 * HID),
                  W_r1, b_r1.reshape(1, HID), W_r2, b_r2.reshape(1, HID),
                  W_r3, b_r3.reshape(1, 1))
    return out


# speedup vs baseline: 3.7312x; 3.7312x over previous
import functools

import jax
import jax.numpy as jnp
from jax import lax
from jax.experimental import pallas as pl
from jax.experimental.pallas import tpu as pltpu
from jax.experimental.pallas import tpu_sc as plsc

N_NODES = 50000
N_EDGES = 800000
N_GRAPHS = 256
IN_DIM = 74
HID = 256
EHID = 32

NP = 51200
GPAD = 384
E_PAD = 802816
EPT = 25088
EPT_SC = 50176
EBATCH = 12544
GID_CH = 1664
FB = 128

_i32 = jnp.int32
_f32 = jnp.float32


def _iota16():
    return lax.iota(_i32, 16)


def _mesh():
    return plsc.VectorSubcoreMesh(core_axis_name="c", subcore_axis_name="s")


def _copy128(src_ref, src_off, dst_ref):
    for t in range(8):
        dst_ref[16 * t:16 * t + 16] = src_ref[pl.ds(src_off + 16 * t, 16)]


def _hist_scan(ibuf, dmabuf, ones_v, hist, n_idx, sent_base):
    nb_full = n_idx // FB
    rem = n_idx - nb_full * FB

    def body(j, carry):
        _copy128(ibuf, FB * j, dmabuf)
        pltpu.sync_copy(ones_v, hist.at[dmabuf], add=True)
        return carry

    lax.fori_loop(0, nb_full, body, jnp.int32(0))
    if rem > 0:
        for t in range(8):
            lane0 = 16 * t
            v = ibuf[pl.ds(FB * nb_full + lane0, 16)]
            sent = sent_base + _iota16()
            if lane0 + 16 <= rem:
                dmabuf[16 * t:16 * t + 16] = v
            elif lane0 >= rem:
                dmabuf[16 * t:16 * t + 16] = sent
            else:
                m = _iota16() < (rem - lane0)
                dmabuf[16 * t:16 * t + 16] = jnp.where(m, v, sent)
        pltpu.sync_copy(ones_v, hist.at[dmabuf], add=True)


def _deg_kernel(src_hbm, dst_hbm, gid_hbm, zflat_hbm,
                dop, dip, cntp,
                hist_o, hist_i, hist_g,
                ibuf, dmabuf, ones_v):
    sc = lax.axis_index("c")
    sub = lax.axis_index("s")
    for t in range(8):
        ones_v[16 * t:16 * t + 16] = jnp.ones((16,), _f32)
    pltpu.sync_copy(zflat_hbm, hist_o.at[pl.ds(sub * 3200, 3200)])
    pltpu.sync_copy(zflat_hbm, hist_i.at[pl.ds(sub * 3200, 3200)])

    @pl.when(sub == 0)
    def _():
        pltpu.sync_copy(zflat_hbm.at[pl.ds(0, 256)], hist_g.at[pl.ds(0, 256)])

    plsc.subcore_barrier()

    ebase = (sc * 16 + sub) * EPT
    pltpu.sync_copy(src_hbm.at[pl.ds(ebase, EPT)], ibuf.at[pl.ds(0, EPT)])
    _hist_scan(ibuf, dmabuf, ones_v, hist_o, EPT, NP)
    pltpu.sync_copy(dst_hbm.at[pl.ds(ebase, EPT)], ibuf.at[pl.ds(0, EPT)])
    _hist_scan(ibuf, dmabuf, ones_v, hist_i, EPT, NP)

    gbase = (sc * 16 + sub) * GID_CH
    pltpu.sync_copy(gid_hbm.at[pl.ds(gbase, GID_CH)], ibuf.at[pl.ds(0, GID_CH)])
    _hist_scan(ibuf, dmabuf, ones_v, hist_g, GID_CH, N_GRAPHS)

    plsc.subcore_barrier()
    pltpu.sync_copy(hist_o.at[pl.ds(sub * 3200, 3200)],
                    dop.at[pl.ds(sc * NP + sub * 3200, 3200)])
    pltpu.sync_copy(hist_i.at[pl.ds(sub * 3200, 3200)],
                    dip.at[pl.ds(sc * NP + sub * 3200, 3200)])

    @pl.when(sub == 0)
    def _():
        pltpu.sync_copy(hist_g, cntp.at[pl.ds(sc * GPAD, GPAD)])


def _run_deg(src, dst, gid_pad, zflat):
    k = functools.partial(
        pl.kernel,
        out_type=[
            jax.ShapeDtypeStruct((2 * NP,), _f32),
            jax.ShapeDtypeStruct((2 * NP,), _f32),
            jax.ShapeDtypeStruct((2 * GPAD,), _f32),
        ],
        mesh=_mesh(),
        compiler_params=pltpu.CompilerParams(needs_layout_passes=False),
        scratch_types=[
            pltpu.VMEM_SHARED((NP + 16,), _f32),
            pltpu.VMEM_SHARED((NP + 16,), _f32),
            pltpu.VMEM_SHARED((GPAD,), _f32),
            pltpu.VMEM((FB * (EPT // FB) + FB,), _i32),
            pltpu.VMEM((FB,), _i32),
            pltpu.VMEM((FB,), _f32),
        ],
    )(_deg_kernel)
    return k(src, dst, gid_pad, zflat)


def _make_agg_kernel(R, n_chunks, ebatch, quota, fb, split):
    cpc = (n_chunks + 1) // 2
    n_batches = quota // ebatch
    vregs = ebatch // 16
    arows = split * R
    rpt = arows // 16
    fbv = fb // 16

    def flush(table_hbm, acc, csrc_st, cdst_st, csrc_dma, cdst_dma,
              rowbuf, sem, off):
        for t in range(fbv):
            csrc_dma[16 * t:16 * t + 16] = csrc_st[16 * t:16 * t + 16]
            cdst_dma[16 * t:16 * t + 16] = cdst_st[16 * t:16 * t + 16]
        pltpu.async_copy(table_hbm.at[csrc_dma], rowbuf, sem).wait()
        pltpu.sync_copy(rowbuf, acc.at[cdst_dma], add=True)
        for t in range(2):
            ts = csrc_st[pl.ds(fb + 16 * t, 16)]
            td = cdst_st[pl.ds(fb + 16 * t, 16)]
            csrc_st[16 * t:16 * t + 16] = ts
            cdst_st[16 * t:16 * t + 16] = td
        return off - fb

    def body(table_hbm, src_hbm, dst_hbm, zr_hbm, out_hbm,
             acc, sbuf, dbuf, csrc_st, cdst_st, csrc_dma, cdst_dma,
             rowbuf, sem):
        sc = lax.axis_index("c")
        sub = lax.axis_index("s")
        ebase0 = sub * quota

        for i in range(cpc):
            c = 2 * i + sc

            @pl.when(c < n_chunks)
            def _chunk():
                lo = c * R
                hi = lo + R

                pltpu.sync_copy(zr_hbm, rowbuf.at[pl.ds(0, 16)])

                def zbody(jj, carry):
                    pltpu.sync_copy(rowbuf.at[pl.ds(0, 16)],
                                    acc.at[pl.ds(sub * rpt + 16 * jj, 16)])
                    return carry
                lax.fori_loop(0, rpt // 16, zbody, jnp.int32(0))

                @pl.when(sub == 0)
                def _():
                    for t in range(split):
                        pltpu.sync_copy(rowbuf.at[pl.ds(0, 16)],
                                        acc.at[pl.ds(arows + 16 * t, 16)])

                plsc.subcore_barrier()

                off = jnp.int32(0)
                for b in range(n_batches):
                    eb = ebase0 + b * ebatch
                    pltpu.sync_copy(src_hbm.at[pl.ds(eb, ebatch)], sbuf)
                    pltpu.sync_copy(dst_hbm.at[pl.ds(eb, ebatch)], dbuf)

                    def vbody(v, o):
                        sv = sbuf[pl.ds(v * 16, 16)]
                        d = dbuf[pl.ds(v * 16, 16)]
                        m = (d >= lo) & (d < hi)
                        mi = jnp.where(m, 1, 0).astype(_i32)
                        csum = plsc.cumsum(mi)
                        if split == 1:
                            pos = o + csum - 1
                            plsc.store_scatter(csrc_st, [pos], sv, mask=m)
                            plsc.store_scatter(cdst_st, [pos], d - lo, mask=m)
                            o = o + jnp.sum(mi)
                        else:
                            pos = o + 2 * csum - 2
                            s2 = 2 * sv
                            d2 = 2 * (d - lo)
                            plsc.store_scatter(csrc_st, [pos], s2, mask=m)
                            plsc.store_scatter(cdst_st, [pos], d2, mask=m)
                            plsc.store_scatter(csrc_st, [pos + 1], s2 + 1,
                                               mask=m)
                            plsc.store_scatter(cdst_st, [pos + 1], d2 + 1,
                                               mask=m)
                            o = o + 2 * jnp.sum(mi)
                        return lax.cond(
                            o >= fb,
                            lambda oo: flush(table_hbm, acc, csrc_st, cdst_st,
                                             csrc_dma, cdst_dma, rowbuf, sem,
                                             oo),
                            lambda oo: oo,
                            o)

                    off = lax.fori_loop(0, vregs, vbody, off)

                for t in range(fbv):
                    lane0 = 16 * t
                    mv = (lane0 + _iota16()) < off
                    sv = csrc_st[16 * t:16 * t + 16]
                    dv = cdst_st[16 * t:16 * t + 16]
                    csrc_dma[16 * t:16 * t + 16] = jnp.where(mv, sv, _iota16())
                    cdst_dma[16 * t:16 * t + 16] = jnp.where(
                        mv, dv, arows + _iota16())
                pltpu.async_copy(table_hbm.at[csrc_dma], rowbuf, sem).wait()
                pltpu.sync_copy(rowbuf, acc.at[cdst_dma], add=True)

                plsc.subcore_barrier()
                pltpu.sync_copy(
                    acc.at[pl.ds(sub * rpt, rpt)],
                    out_hbm.at[pl.ds(c * arows + sub * rpt, rpt)])
                plsc.subcore_barrier()

    return body


def _run_agg(table, src, dst, zr, R, n_chunks, quota, ebatch, fb=96, split=1):
    body = _make_agg_kernel(R, n_chunks, ebatch, quota, fb, split)
    k = functools.partial(
        pl.kernel,
        out_type=jax.ShapeDtypeStruct((n_chunks * R * split, 128), _f32),
        mesh=_mesh(),
        compiler_params=pltpu.CompilerParams(needs_layout_passes=False),
        scratch_types=[
            pltpu.VMEM_SHARED((split * (R + 16), 128), _f32),
            pltpu.VMEM((ebatch,), _i32),
            pltpu.VMEM((ebatch,), _i32),
            pltpu.VMEM((fb + 32,), _i32),
            pltpu.VMEM((fb + 32,), _i32),
            pltpu.VMEM((fb,), _i32),
            pltpu.VMEM((fb,), _i32),
            pltpu.VMEM((fb, 128), _f32),
            pltpu.SemaphoreType.DMA,
        ],
    )(body)
    return k(table, src, dst, zr)


_BLK = 512
_NBLK = NP // _BLK


def _t1_body(nf_ref, do0, do1, di0, di1, xn_ref, no_ref, ni_ref):
    do = do0[...] + do1[...]
    di = di0[...] + di1[...]
    no = jnp.where(do > 0.0, lax.rsqrt(jnp.maximum(do, 1e-30)), 0.0)
    ni = jnp.where(di > 0.0, lax.rsqrt(jnp.maximum(di, 1e-30)), 0.0)
    xn_ref[...] = nf_ref[...] * no[:, None]
    no_ref[...] = no
    ni_ref[...] = ni


def _run_t1(nf_p, dop, dip):
    return pl.pallas_call(
        _t1_body,
        grid=(_NBLK,),
        in_specs=[
            pl.BlockSpec((_BLK, 128), lambda i: (i, 0)),
            pl.BlockSpec((_BLK,), lambda i: (i,)),
            pl.BlockSpec((_BLK,), lambda i: (i,)),
            pl.BlockSpec((_BLK,), lambda i: (i,)),
            pl.BlockSpec((_BLK,), lambda i: (i,)),
        ],
        out_specs=[
            pl.BlockSpec((_BLK, 128), lambda i: (i, 0)),
            pl.BlockSpec((_BLK,), lambda i: (i,)),
            pl.BlockSpec((_BLK,), lambda i: (i,)),
        ],
        out_shape=[
            jax.ShapeDtypeStruct((NP, 128), _f32),
            jax.ShapeDtypeStruct((NP,), _f32),
            jax.ShapeDtypeStruct((NP,), _f32),
        ],
    )(nf_p, dop[:NP], dop[NP:], dip[:NP], dip[NP:])


def _t2_body(agg_ref, ni_ref, no_ref, w_ref, b_ref, out_ref):
    a = agg_ref[...] * ni_ref[...][:, None]
    h = jnp.maximum(jnp.dot(a, w_ref[...],
                            preferred_element_type=_f32) + b_ref[...], 0.0)
    out_ref[...] = h * no_ref[...][:, None]


def _run_t2(agg1, ni, no, w, b):
    return pl.pallas_call(
        _t2_body,
        grid=(_NBLK,),
        in_specs=[
            pl.BlockSpec((_BLK, 128), lambda i: (i, 0)),
            pl.BlockSpec((_BLK,), lambda i: (i,)),
            pl.BlockSpec((_BLK,), lambda i: (i,)),
            pl.BlockSpec((128, HID), lambda i: (0, 0)),
            pl.BlockSpec((1, HID), lambda i: (0, 0)),
        ],
        out_specs=pl.BlockSpec((_BLK, HID), lambda i: (i, 0)),
        out_shape=jax.ShapeDtypeStruct((NP, HID), _f32),
    )(agg1, ni, no, w, b)


def _t3_body(agg_ref, ni_ref, w_ref, b_ref, out_ref):
    a = agg_ref[...] * ni_ref[...][:, None]
    out_ref[...] = jnp.maximum(
        jnp.dot(a, w_ref[...], preferred_element_type=_f32) + b_ref[...], 0.0)


def _run_t3(agg2, ni, w, b):
    return pl.pallas_call(
        _t3_body,
        grid=(_NBLK,),
        in_specs=[
            pl.BlockSpec((_BLK, HID), lambda i: (i, 0)),
            pl.BlockSpec((_BLK,), lambda i: (i,)),
            pl.BlockSpec((HID, HID), lambda i: (0, 0)),
            pl.BlockSpec((1, HID), lambda i: (0, 0)),
        ],
        out_specs=pl.BlockSpec((_BLK, HID), lambda i: (i, 0)),
        out_shape=jax.ShapeDtypeStruct((NP, HID), _f32),
    )(agg2, ni, w, b)


def _t4_body(pp_ref, c0_ref, c1_ref, ihb_ref, iff_ref, ssrc_ref, sdst_ref,
             wproj_ref, bproj_ref, we1_ref, be1_ref, w2r_ref, be2r_ref,
             bnn_ref, wih_ref, whh_ref, bih_ref, bhh_ref,
             wr1_ref, br1_ref, wr2_ref, br2_ref, wr3_ref, br3_ref,
             out_ref):
    pool = pp_ref[...]
    cnt = (c0_ref[...] + c1_ref[...])[:N_GRAPHS]
    g = pool[:N_GRAPHS] / jnp.clip(cnt, 1.0, None)[:, None]
    gm = jnp.concatenate([g, ihb_ref[...]], axis=1)
    gm2 = jnp.concatenate([gm, gm], axis=0)
    nf = jnp.maximum(
        jnp.dot(gm2, wproj_ref[...], preferred_element_type=_f32)
        + bproj_ref[...], 0.0)
    eh = jnp.maximum(
        jnp.dot(iff_ref[...], we1_ref[...], preferred_element_type=_f32)
        + be1_ref[...], 0.0)
    iot = lax.broadcasted_iota(_i32, (4 * N_GRAPHS, 2 * N_GRAPHS), 1)
    oh_s = (ssrc_ref[...] == iot).astype(_f32)
    nfs = jnp.dot(oh_s, nf, preferred_element_type=_f32)
    msg = jnp.dot(nfs, be2r_ref[...], preferred_element_type=_f32)
    for kk in range(EHID):
        msg = msg + eh[:, kk:kk + 1] * jnp.dot(
            nfs, w2r_ref[kk], preferred_element_type=_f32)
    oh_d = (sdst_ref[...] == iot).astype(_f32)
    aggm = lax.dot_general(oh_d, msg, (((0,), (0,)), ((), ())),
                           preferred_element_type=_f32)
    nf2 = jnp.maximum(aggm + bnn_ref[...], 0.0)
    gi = lax.dot_general(nf2, wih_ref[...], (((1,), (1,)), ((), ())),
                         preferred_element_type=_f32) + bih_ref[...]
    gh = lax.dot_general(nf, whh_ref[...], (((1,), (1,)), ((), ())),
                         preferred_element_type=_f32) + bhh_ref[...]
    r = jax.nn.sigmoid(gi[:, :HID] + gh[:, :HID])
    z = jax.nn.sigmoid(gi[:, HID:2 * HID] + gh[:, HID:2 * HID])
    n = jnp.tanh(gi[:, 2 * HID:] + r * gh[:, 2 * HID:])
    ghf = (1.0 - z) * n + z * nf
    o = jnp.maximum(jnp.dot(ghf, wr1_ref[...],
                            preferred_element_type=_f32) + br1_ref[...], 0.0)
    o = jnp.maximum(jnp.dot(o, wr2_ref[...],
                            preferred_element_type=_f32) + br2_ref[...], 0.0)
    o = jnp.dot(o, wr3_ref[...], preferred_element_type=_f32) + br3_ref[...]
    out_ref[...] = 0.5 * (o[:N_GRAPHS] + o[N_GRAPHS:])


def _run_t4(pp, cntp, ihb, iff, ssrc, sdst, wproj, bproj, we1, be1, w2r,
            be2r, bnn, wih, whh, bih, bhh, wr1, br1, wr2, br2, wr3, br3):
    return pl.pallas_call(
        _t4_body,
        out_shape=jax.ShapeDtypeStruct((N_GRAPHS, 1), _f32),
    )(pp, cntp[:GPAD], cntp[GPAD:], ihb, iff, ssrc, sdst, wproj, bproj,
      we1, be1, w2r, be2r, bnn, wih, whh, bih, bhh, wr1, br1, wr2, br2,
      wr3, br3)


def kernel(node_feats, edge_index, graph_id, inter_hb, be_salt, be_ps, ip,
           solv_edge_index, W_c1, b_c1, W_c2, b_c2, W_proj, b_proj,
           W_e1, b_e1, W_e2, b_e2, b_nn, W_ih, W_hh, b_ih, b_hh,
           W_r1, b_r1, W_r2, b_r2, W_r3, b_r3):
    epad = NP + (jnp.arange(E_PAD - N_EDGES, dtype=_i32) % 16)
    src = jnp.concatenate([edge_index[0], epad])
    dst = jnp.concatenate([edge_index[1], epad])
    gid_np = jnp.pad(graph_id, (0, NP - N_NODES), constant_values=N_GRAPHS)
    gid_pad = jnp.pad(gid_np.reshape(32, 1600), ((0, 0), (0, GID_CH - 1600)),
                      constant_values=N_GRAPHS).reshape(-1)
    nf_p = jnp.pad(node_feats, ((0, NP - N_NODES), (0, 128 - IN_DIM)))
    wc1_p = jnp.pad(W_c1, ((0, 128 - IN_DIM), (0, 0)))
    zflat = jnp.zeros((3200,), _f32)
    zr128 = jnp.zeros((16, 128), _f32)

    dop, dip, cntp = _run_deg(src, dst, gid_pad, zflat)
    xn, no, ni = _run_t1(nf_p, dop, dip)
    agg1 = _run_agg(xn, src, dst, zr128, 12800, 4, EPT_SC, 6272)
    h1n = _run_t2(agg1, ni, no, wc1_p, b_c1.reshape(1, HID))
    agg2 = _run_agg(h1n.reshape(2 * NP, 128), src, dst, zr128,
                    5120, 10, EPT_SC, 6272, split=2).reshape(NP, HID)
    h2 = _run_t3(agg2, ni, W_c2, b_c2.reshape(1, HID))
    psrc = jnp.arange(NP, dtype=_i32)
    pp = _run_agg(h2.reshape(2 * NP, 128), psrc, gid_np, zr128,
                  GPAD, 1, NP // 16, 3200, split=2).reshape(GPAD, HID)

    iff = jnp.concatenate([inter_hb, be_salt, be_ps, ip], axis=0)[:, None]
    ssrc = solv_edge_index[0][:, None]
    sdst = solv_edge_index[1][:, None]
    out = _run_t4(pp, cntp, inter_hb[:, None], iff, ssrc, sdst,
                  W_proj, b_proj.reshape(1, HID),
                  W_e1, b_e1.reshape(1, EHID),
                  W_e2.reshape(EHID, HID, HID), b_e2.reshape(HID, HID),
                  b_nn.reshape(1, HID), W_ih, W_hh,
                  b_ih.reshape(1, 3 * HID), b_hh.reshape(1, 3 * HID),
                  W_r1, b_r1.reshape(1, HID), W_r2, b_r2.reshape(1, HID),
                  W_r3, b_r3.reshape(1, 1))
    return out


# speedup vs baseline: 3.9162x; 1.0496x over previous
import functools

import jax
import jax.numpy as jnp
from jax import lax
from jax.experimental import pallas as pl
from jax.experimental.pallas import tpu as pltpu
from jax.experimental.pallas import tpu_sc as plsc

N_NODES = 50000
N_EDGES = 800000
N_GRAPHS = 256
IN_DIM = 74
HID = 256
EHID = 32

NP = 51200
GPAD = 384
E_PAD = 802816
EPT = 25088
EPT_SC = 50176
EBATCH = 12544
GID_CH = 1664
FB = 128

_i32 = jnp.int32
_f32 = jnp.float32


def _iota16():
    return lax.iota(_i32, 16)


def _mesh():
    return plsc.VectorSubcoreMesh(core_axis_name="c", subcore_axis_name="s")


def _copy128(src_ref, src_off, dst_ref):
    for t in range(8):
        dst_ref[16 * t:16 * t + 16] = src_ref[pl.ds(src_off + 16 * t, 16)]


def _hist_scan(ibuf, dmabuf, ones_v, hist, n_idx, sent_base):
    nb_full = n_idx // FB
    rem = n_idx - nb_full * FB

    def body(j, carry):
        _copy128(ibuf, FB * j, dmabuf)
        pltpu.sync_copy(ones_v, hist.at[dmabuf], add=True)
        return carry

    lax.fori_loop(0, nb_full, body, jnp.int32(0))
    if rem > 0:
        for t in range(8):
            lane0 = 16 * t
            v = ibuf[pl.ds(FB * nb_full + lane0, 16)]
            sent = sent_base + _iota16()
            if lane0 + 16 <= rem:
                dmabuf[16 * t:16 * t + 16] = v
            elif lane0 >= rem:
                dmabuf[16 * t:16 * t + 16] = sent
            else:
                m = _iota16() < (rem - lane0)
                dmabuf[16 * t:16 * t + 16] = jnp.where(m, v, sent)
        pltpu.sync_copy(ones_v, hist.at[dmabuf], add=True)


def _deg_kernel(src_hbm, dst_hbm, gid_hbm, zflat_hbm,
                dop, dip, cntp,
                hist_o, hist_i, hist_g,
                ibuf, dmabuf, ones_v):
    sc = lax.axis_index("c")
    sub = lax.axis_index("s")
    for t in range(8):
        ones_v[16 * t:16 * t + 16] = jnp.ones((16,), _f32)
    pltpu.sync_copy(zflat_hbm, hist_o.at[pl.ds(sub * 3200, 3200)])
    pltpu.sync_copy(zflat_hbm, hist_i.at[pl.ds(sub * 3200, 3200)])

    @pl.when(sub == 0)
    def _():
        pltpu.sync_copy(zflat_hbm.at[pl.ds(0, 256)], hist_g.at[pl.ds(0, 256)])

    plsc.subcore_barrier()

    ebase = (sc * 16 + sub) * EPT
    pltpu.sync_copy(src_hbm.at[pl.ds(ebase, EPT)], ibuf.at[pl.ds(0, EPT)])
    _hist_scan(ibuf, dmabuf, ones_v, hist_o, EPT, NP)
    pltpu.sync_copy(dst_hbm.at[pl.ds(ebase, EPT)], ibuf.at[pl.ds(0, EPT)])
    _hist_scan(ibuf, dmabuf, ones_v, hist_i, EPT, NP)

    gbase = (sc * 16 + sub) * GID_CH
    pltpu.sync_copy(gid_hbm.at[pl.ds(gbase, GID_CH)], ibuf.at[pl.ds(0, GID_CH)])
    _hist_scan(ibuf, dmabuf, ones_v, hist_g, GID_CH, N_GRAPHS)

    plsc.subcore_barrier()
    pltpu.sync_copy(hist_o.at[pl.ds(sub * 3200, 3200)],
                    dop.at[pl.ds(sc * NP + sub * 3200, 3200)])
    pltpu.sync_copy(hist_i.at[pl.ds(sub * 3200, 3200)],
                    dip.at[pl.ds(sc * NP + sub * 3200, 3200)])

    @pl.when(sub == 0)
    def _():
        pltpu.sync_copy(hist_g, cntp.at[pl.ds(sc * GPAD, GPAD)])


def _run_deg(src, dst, gid_pad, zflat):
    k = functools.partial(
        pl.kernel,
        out_type=[
            jax.ShapeDtypeStruct((2 * NP,), _f32),
            jax.ShapeDtypeStruct((2 * NP,), _f32),
            jax.ShapeDtypeStruct((2 * GPAD,), _f32),
        ],
        mesh=_mesh(),
        compiler_params=pltpu.CompilerParams(needs_layout_passes=False),
        scratch_types=[
            pltpu.VMEM_SHARED((NP + 16,), _f32),
            pltpu.VMEM_SHARED((NP + 16,), _f32),
            pltpu.VMEM_SHARED((GPAD,), _f32),
            pltpu.VMEM((FB * (EPT // FB) + FB,), _i32),
            pltpu.VMEM((FB,), _i32),
            pltpu.VMEM((FB,), _f32),
        ],
    )(_deg_kernel)
    return k(src, dst, gid_pad, zflat)


def _make_agg_kernel(R, n_chunks, ebatch, quota, fb, split):
    cpc = (n_chunks + 1) // 2
    n_batches = quota // ebatch
    vregs = ebatch // 16
    arows = split * R
    rpt = arows // 16
    fbv = fb // 16

    def flush(table_hbm, acc, csrc_st, cdst_st, csrc_dma, cdst_dma,
              rowbuf, sem, off):
        for t in range(fbv):
            csrc_dma[16 * t:16 * t + 16] = csrc_st[16 * t:16 * t + 16]
            cdst_dma[16 * t:16 * t + 16] = cdst_st[16 * t:16 * t + 16]
        pltpu.async_copy(table_hbm.at[csrc_dma], rowbuf, sem).wait()
        pltpu.sync_copy(rowbuf, acc.at[cdst_dma], add=True)
        for t in range(2):
            ts = csrc_st[pl.ds(fb + 16 * t, 16)]
            td = cdst_st[pl.ds(fb + 16 * t, 16)]
            csrc_st[16 * t:16 * t + 16] = ts
            cdst_st[16 * t:16 * t + 16] = td
        return off - fb

    def body(table_hbm, src_hbm, dst_hbm, zr_hbm, out_hbm,
             acc, sbuf, dbuf, csrc_st, cdst_st, csrc_dma, cdst_dma,
             rowbuf, sem):
        sc = lax.axis_index("c")
        sub = lax.axis_index("s")
        ebase0 = sub * quota

        for i in range(cpc):
            c = 2 * i + sc

            @pl.when(c < n_chunks)
            def _chunk():
                lo = c * R
                hi = lo + R

                pltpu.sync_copy(zr_hbm, rowbuf.at[pl.ds(0, 16)])

                def zbody(jj, carry):
                    pltpu.sync_copy(rowbuf.at[pl.ds(0, 16)],
                                    acc.at[pl.ds(sub * rpt + 16 * jj, 16)])
                    return carry
                lax.fori_loop(0, rpt // 16, zbody, jnp.int32(0))

                @pl.when(sub == 0)
                def _():
                    for t in range(split):
                        pltpu.sync_copy(rowbuf.at[pl.ds(0, 16)],
                                        acc.at[pl.ds(arows + 16 * t, 16)])

                plsc.subcore_barrier()

                off = jnp.int32(0)
                for b in range(n_batches):
                    eb = ebase0 + b * ebatch
                    pltpu.sync_copy(src_hbm.at[pl.ds(eb, ebatch)], sbuf)
                    pltpu.sync_copy(dst_hbm.at[pl.ds(eb, ebatch)], dbuf)

                    def vbody(v, o):
                        sv = sbuf[pl.ds(v * 16, 16)]
                        d = dbuf[pl.ds(v * 16, 16)]
                        m = (d >= lo) & (d < hi)
                        mi = jnp.where(m, 1, 0).astype(_i32)
                        csum = plsc.cumsum(mi)
                        if split == 1:
                            pos = o + csum - 1
                            plsc.store_scatter(csrc_st, [pos], sv, mask=m)
                            plsc.store_scatter(cdst_st, [pos], d - lo, mask=m)
                            o = o + jnp.sum(mi)
                        else:
                            pos = o + 2 * csum - 2
                            s2 = 2 * sv
                            d2 = 2 * (d - lo)
                            plsc.store_scatter(csrc_st, [pos], s2, mask=m)
                            plsc.store_scatter(cdst_st, [pos], d2, mask=m)
                            plsc.store_scatter(csrc_st, [pos + 1], s2 + 1,
                                               mask=m)
                            plsc.store_scatter(cdst_st, [pos + 1], d2 + 1,
                                               mask=m)
                            o = o + 2 * jnp.sum(mi)
                        return lax.cond(
                            o >= fb,
                            lambda oo: flush(table_hbm, acc, csrc_st, cdst_st,
                                             csrc_dma, cdst_dma, rowbuf, sem,
                                             oo),
                            lambda oo: oo,
                            o)

                    off = lax.fori_loop(0, vregs, vbody, off)

                for t in range(fbv):
                    lane0 = 16 * t
                    mv = (lane0 + _iota16()) < off
                    sv = csrc_st[16 * t:16 * t + 16]
                    dv = cdst_st[16 * t:16 * t + 16]
                    csrc_dma[16 * t:16 * t + 16] = jnp.where(mv, sv, _iota16())
                    cdst_dma[16 * t:16 * t + 16] = jnp.where(
                        mv, dv, arows + _iota16())
                pltpu.async_copy(table_hbm.at[csrc_dma], rowbuf, sem).wait()
                pltpu.sync_copy(rowbuf, acc.at[cdst_dma], add=True)

                plsc.subcore_barrier()
                pltpu.sync_copy(
                    acc.at[pl.ds(sub * rpt, rpt)],
                    out_hbm.at[pl.ds(c * arows + sub * rpt, rpt)])
                plsc.subcore_barrier()

    return body


def _run_agg(table, src, dst, zr, R, n_chunks, quota, ebatch, fb=96, split=1):
    body = _make_agg_kernel(R, n_chunks, ebatch, quota, fb, split)
    k = functools.partial(
        pl.kernel,
        out_type=jax.ShapeDtypeStruct((n_chunks * R * split, 128), _f32),
        mesh=_mesh(),
        compiler_params=pltpu.CompilerParams(needs_layout_passes=False),
        scratch_types=[
            pltpu.VMEM_SHARED((split * (R + 16), 128), _f32),
            pltpu.VMEM((ebatch,), _i32),
            pltpu.VMEM((ebatch,), _i32),
            pltpu.VMEM((fb + 32,), _i32),
            pltpu.VMEM((fb + 32,), _i32),
            pltpu.VMEM((fb,), _i32),
            pltpu.VMEM((fb,), _i32),
            pltpu.VMEM((fb, 128), _f32),
            pltpu.SemaphoreType.DMA,
        ],
    )(body)
    return k(table, src, dst, zr)


_BLK = 512
_NBLK = NP // _BLK


def _t1_body(nf_ref, do0, do1, di0, di1, xn_ref, no_ref, ni_ref):
    do = do0[...] + do1[...]
    di = di0[...] + di1[...]
    no = jnp.where(do > 0.0, lax.rsqrt(jnp.maximum(do, 1e-30)), 0.0)
    ni = jnp.where(di > 0.0, lax.rsqrt(jnp.maximum(di, 1e-30)), 0.0)
    xn_ref[...] = nf_ref[...] * no[:, None]
    no_ref[...] = no
    ni_ref[...] = ni


def _run_t1(nf_p, dop, dip):
    return pl.pallas_call(
        _t1_body,
        grid=(_NBLK,),
        in_specs=[
            pl.BlockSpec((_BLK, 128), lambda i: (i, 0)),
            pl.BlockSpec((_BLK,), lambda i: (i,)),
            pl.BlockSpec((_BLK,), lambda i: (i,)),
            pl.BlockSpec((_BLK,), lambda i: (i,)),
            pl.BlockSpec((_BLK,), lambda i: (i,)),
        ],
        out_specs=[
            pl.BlockSpec((_BLK, 128), lambda i: (i, 0)),
            pl.BlockSpec((_BLK,), lambda i: (i,)),
            pl.BlockSpec((_BLK,), lambda i: (i,)),
        ],
        out_shape=[
            jax.ShapeDtypeStruct((NP, 128), _f32),
            jax.ShapeDtypeStruct((NP,), _f32),
            jax.ShapeDtypeStruct((NP,), _f32),
        ],
    )(nf_p, dop[:NP], dop[NP:], dip[:NP], dip[NP:])


def _t2_body(agg_ref, ni_ref, no_ref, w_ref, b_ref, out_ref):
    a = agg_ref[...] * ni_ref[...][:, None]
    h = jnp.maximum(jnp.dot(a, w_ref[...],
                            preferred_element_type=_f32) + b_ref[...], 0.0)
    out_ref[...] = h * no_ref[...][:, None]


def _run_t2(agg1, ni, no, w, b):
    return pl.pallas_call(
        _t2_body,
        grid=(_NBLK,),
        in_specs=[
            pl.BlockSpec((_BLK, 128), lambda i: (i, 0)),
            pl.BlockSpec((_BLK,), lambda i: (i,)),
            pl.BlockSpec((_BLK,), lambda i: (i,)),
            pl.BlockSpec((128, HID), lambda i: (0, 0)),
            pl.BlockSpec((1, HID), lambda i: (0, 0)),
        ],
        out_specs=pl.BlockSpec((_BLK, HID), lambda i: (i, 0)),
        out_shape=jax.ShapeDtypeStruct((NP, HID), _f32),
    )(agg1, ni, no, w, b)


def _t3_body(agg_ref, ni_ref, w_ref, b_ref, out_ref):
    a = agg_ref[...] * ni_ref[...][:, None]
    out_ref[...] = jnp.maximum(
        jnp.dot(a, w_ref[...], preferred_element_type=_f32) + b_ref[...], 0.0)


def _run_t3(agg2, ni, w, b):
    return pl.pallas_call(
        _t3_body,
        grid=(_NBLK,),
        in_specs=[
            pl.BlockSpec((_BLK, HID), lambda i: (i, 0)),
            pl.BlockSpec((_BLK,), lambda i: (i,)),
            pl.BlockSpec((HID, HID), lambda i: (0, 0)),
            pl.BlockSpec((1, HID), lambda i: (0, 0)),
        ],
        out_specs=pl.BlockSpec((_BLK, HID), lambda i: (i, 0)),
        out_shape=jax.ShapeDtypeStruct((NP, HID), _f32),
    )(agg2, ni, w, b)


def _t4_body(pp_ref, c0_ref, c1_ref, ihb_ref, iff_ref, ssrc_ref, sdst_ref,
             wproj_ref, bproj_ref, we1_ref, be1_ref, w2r_ref, be2r_ref,
             bnn_ref, wih_ref, whh_ref, bih_ref, bhh_ref,
             wr1_ref, br1_ref, wr2_ref, br2_ref, wr3_ref, br3_ref,
             out_ref):
    pool = pp_ref[...]
    cnt = (c0_ref[...] + c1_ref[...])[:N_GRAPHS]
    g = pool[:N_GRAPHS] / jnp.clip(cnt, 1.0, None)[:, None]
    gm = jnp.concatenate([g, ihb_ref[...]], axis=1)
    gm2 = jnp.concatenate([gm, gm], axis=0)
    nf = jnp.maximum(
        jnp.dot(gm2, wproj_ref[...], preferred_element_type=_f32)
        + bproj_ref[...], 0.0)
    eh = jnp.maximum(
        jnp.dot(iff_ref[...], we1_ref[...], preferred_element_type=_f32)
        + be1_ref[...], 0.0)
    iot = lax.broadcasted_iota(_i32, (4 * N_GRAPHS, 2 * N_GRAPHS), 1)
    oh_s = (ssrc_ref[...] == iot).astype(_f32)
    nfs = jnp.dot(oh_s, nf, preferred_element_type=_f32)
    msg = jnp.dot(nfs, be2r_ref[...], preferred_element_type=_f32)
    for kk in range(EHID):
        msg = msg + eh[:, kk:kk + 1] * jnp.dot(
            nfs, w2r_ref[kk], preferred_element_type=_f32)
    oh_d = (sdst_ref[...] == iot).astype(_f32)
    aggm = lax.dot_general(oh_d, msg, (((0,), (0,)), ((), ())),
                           preferred_element_type=_f32)
    nf2 = jnp.maximum(aggm + bnn_ref[...], 0.0)
    gi = lax.dot_general(nf2, wih_ref[...], (((1,), (1,)), ((), ())),
                         preferred_element_type=_f32) + bih_ref[...]
    gh = lax.dot_general(nf, whh_ref[...], (((1,), (1,)), ((), ())),
                         preferred_element_type=_f32) + bhh_ref[...]
    r = jax.nn.sigmoid(gi[:, :HID] + gh[:, :HID])
    z = jax.nn.sigmoid(gi[:, HID:2 * HID] + gh[:, HID:2 * HID])
    n = jnp.tanh(gi[:, 2 * HID:] + r * gh[:, 2 * HID:])
    ghf = (1.0 - z) * n + z * nf
    o = jnp.maximum(jnp.dot(ghf, wr1_ref[...],
                            preferred_element_type=_f32) + br1_ref[...], 0.0)
    o = jnp.maximum(jnp.dot(o, wr2_ref[...],
                            preferred_element_type=_f32) + br2_ref[...], 0.0)
    o = jnp.dot(o, wr3_ref[...], preferred_element_type=_f32) + br3_ref[...]
    out_ref[...] = 0.5 * (o[:N_GRAPHS] + o[N_GRAPHS:])


def _run_t4(pp, cntp, ihb, iff, ssrc, sdst, wproj, bproj, we1, be1, w2r,
            be2r, bnn, wih, whh, bih, bhh, wr1, br1, wr2, br2, wr3, br3):
    return pl.pallas_call(
        _t4_body,
        out_shape=jax.ShapeDtypeStruct((N_GRAPHS, 1), _f32),
    )(pp, cntp[:GPAD], cntp[GPAD:], ihb, iff, ssrc, sdst, wproj, bproj,
      we1, be1, w2r, be2r, bnn, wih, whh, bih, bhh, wr1, br1, wr2, br2,
      wr3, br3)


def kernel(node_feats, edge_index, graph_id, inter_hb, be_salt, be_ps, ip,
           solv_edge_index, W_c1, b_c1, W_c2, b_c2, W_proj, b_proj,
           W_e1, b_e1, W_e2, b_e2, b_nn, W_ih, W_hh, b_ih, b_hh,
           W_r1, b_r1, W_r2, b_r2, W_r3, b_r3):
    epad = NP + (jnp.arange(E_PAD - N_EDGES, dtype=_i32) % 16)
    src = jnp.concatenate([edge_index[0], epad])
    dst = jnp.concatenate([edge_index[1], epad])
    gid_np = jnp.pad(graph_id, (0, NP - N_NODES), constant_values=N_GRAPHS)
    gid_pad = jnp.pad(gid_np.reshape(32, 1600), ((0, 0), (0, GID_CH - 1600)),
                      constant_values=N_GRAPHS).reshape(-1)
    nf_p = jnp.pad(node_feats, ((0, NP - N_NODES), (0, 128 - IN_DIM)))
    wc1_p = jnp.pad(W_c1, ((0, 128 - IN_DIM), (0, 0)))
    zflat = jnp.zeros((3200,), _f32)
    zr128 = jnp.zeros((16, 128), _f32)

    dop, dip, cntp = _run_deg(src, dst, gid_pad, zflat)
    xn, no, ni = _run_t1(nf_p, dop, dip)
    agg1 = _run_agg(xn, src, dst, zr128, 12800, 4, EPT_SC, 6272)
    h1n = _run_t2(agg1, ni, no, wc1_p, b_c1.reshape(1, HID))
    agg2 = _run_agg(h1n.reshape(2 * NP, 128), src, dst, zr128,
                    6400, 8, EPT_SC, 6272, split=2).reshape(NP, HID)
    h2 = _run_t3(agg2, ni, W_c2, b_c2.reshape(1, HID))
    psrc = jnp.arange(NP, dtype=_i32)
    pp = _run_agg(h2.reshape(2 * NP, 128), psrc, gid_np, zr128,
                  GPAD // 2, 2, NP // 16, 3200, split=2).reshape(GPAD, HID)

    iff = jnp.concatenate([inter_hb, be_salt, be_ps, ip], axis=0)[:, None]
    ssrc = solv_edge_index[0][:, None]
    sdst = solv_edge_index[1][:, None]
    out = _run_t4(pp, cntp, inter_hb[:, None], iff, ssrc, sdst,
                  W_proj, b_proj.reshape(1, HID),
                  W_e1, b_e1.reshape(1, EHID),
                  W_e2.reshape(EHID, HID, HID), b_e2.reshape(HID, HID),
                  b_nn.reshape(1, HID), W_ih, W_hh,
                  b_ih.reshape(1, 3 * HID), b_hh.reshape(1, 3 * HID),
                  W_r1, b_r1.reshape(1, HID), W_r2, b_r2.reshape(1, HID),
                  W_r3, b_r3.reshape(1, 1))
    return out
